# baseline (speedup 1.0000x reference)
import jax
import jax.numpy as jnp
from jax.experimental import pallas as pl
from jax.experimental.pallas import tpu as pltpu
from jax.experimental.pallas import tpu_sc as plsc

NUM_MODELS = 8
STATE_DIM = 256
AC_DIM = 64
HIDDEN = 512
BATCH = 4096
IN_DIM = STATE_DIM + AC_DIM

IN_PAD = 384
R = 256
T_MAX = BATCH // R + NUM_MODELS
PADB = T_MAX * R
WINDOW = 128


def _sc_mesh():
    return plsc.VectorSubcoreMesh(core_axis_name="core",
                                  subcore_axis_name="subcore")


def _sc_scatter_rows(vals, dst, out_rows):
    n, d = vals.shape
    dst2 = dst.reshape(1, n)

    @pl.kernel(out_type=jax.ShapeDtypeStruct((out_rows, d), vals.dtype),
               mesh=_sc_mesh())
    def k(v_hbm, i_hbm, o_hbm):
        def body(v_vmem, i_vmem):
            pltpu.sync_copy(v_vmem, o_hbm.at[i_vmem.at[0]])

        pltpu.emit_pipeline(
            body,
            grid=(n // WINDOW,),
            in_specs=[pl.BlockSpec((WINDOW, d), index_map=lambda i: (i, 0)),
                      pl.BlockSpec((1, WINDOW), index_map=lambda i: (0, i))],
            out_specs=[],
            core_axis_name=("core", "subcore"),
            dimension_semantics=(pltpu.PARALLEL,),
        )(v_hbm, i_hbm)

    return k(vals, dst2)


def _sc_gather_rows(data, g):
    n = g.shape[0]
    d = data.shape[1]
    g2 = g.reshape(1, n)

    @pl.kernel(out_type=jax.ShapeDtypeStruct((n, d), data.dtype),
               mesh=_sc_mesh())
    def k(x_hbm, i_hbm, o_hbm):
        def body(i_vmem, o_vmem):
            pltpu.sync_copy(x_hbm.at[i_vmem.at[0]], o_vmem)

        pltpu.emit_pipeline(
            body,
            grid=(n // WINDOW,),
            in_specs=[pl.BlockSpec((1, WINDOW), index_map=lambda i: (0, i))],
            out_specs=[pl.BlockSpec((WINDOW, d), index_map=lambda i: (i, 0))],
            core_axis_name=("core", "subcore"),
            dimension_semantics=(pltpu.PARALLEL,),
        )(i_hbm, o_hbm)

    return k(data, g2)


_MROWS = 32


def _meta_kernel(idx_ref, slot_ref, te_ref):
    idx2 = idx_ref[...]
    up = (jax.lax.broadcasted_iota(jnp.int32, (128, 128), 0) <
          jax.lax.broadcasted_iota(jnp.int32, (128, 128), 1)
          ).astype(jnp.bfloat16)
    lo = (jax.lax.broadcasted_iota(jnp.int32, (_MROWS, _MROWS), 0) >
          jax.lax.broadcasted_iota(jnp.int32, (_MROWS, _MROWS), 1)
          ).astype(jnp.bfloat16)
    slot = jnp.zeros((_MROWS, 128), jnp.float32)
    ps = jnp.int32(0)
    tile_cum = []
    for e in range(NUM_MODELS):
        m = idx2 == e
        mb = m.astype(jnp.bfloat16)
        lane_pref = jnp.dot(mb, up, preferred_element_type=jnp.float32)
        col_pref = jnp.dot(lo, mb, preferred_element_type=jnp.float32)
        row_off = jnp.sum(col_pref, axis=1, keepdims=True)
        rank = lane_pref + row_off
        count_e = jnp.sum(mb.astype(jnp.float32)).astype(jnp.int32)
        slot = slot + jnp.where(m, rank + ps.astype(jnp.float32), 0.0)
        ps = ps + ((count_e + R - 1) // R) * R
        tile_cum.append(ps // R)
    slot_ref[...] = slot.astype(jnp.int32)
    lane = jax.lax.broadcasted_iota(jnp.int32, (1, 128), 1)
    te = jnp.zeros((1, 128), jnp.int32)
    for e in range(NUM_MODELS):
        te = te + (lane >= tile_cum[e]).astype(jnp.int32)
    te_ref[...] = jnp.minimum(te, NUM_MODELS - 1)


def _routing_meta(idx):
    slot2, te_pad = pl.pallas_call(
        _meta_kernel,
        out_shape=[jax.ShapeDtypeStruct((_MROWS, 128), jnp.int32),
                   jax.ShapeDtypeStruct((1, 128), jnp.int32)],
    )(idx.reshape(_MROWS, 128))
    return slot2.reshape(BATCH), te_pad[0, :T_MAX]


def _cast_kernel(w1_ref, w2_ref, w3_ref, o1_ref, o2_ref, o3_ref):
    o1_ref[...] = w1_ref[...].astype(jnp.bfloat16)
    o2_ref[...] = w2_ref[...].astype(jnp.bfloat16)
    o3_ref[...] = w3_ref[...].astype(jnp.bfloat16)


def _cast_weights(W1, W2, W3):
    spec = lambda k, n: pl.BlockSpec((1, k, n), lambda e: (e, 0, 0))
    return pl.pallas_call(
        _cast_kernel,
        grid=(NUM_MODELS,),
        in_specs=[spec(IN_DIM, HIDDEN), spec(HIDDEN, HIDDEN),
                  spec(HIDDEN, STATE_DIM)],
        out_specs=[spec(IN_DIM, HIDDEN), spec(HIDDEN, HIDDEN),
                   spec(HIDDEN, STATE_DIM)],
        out_shape=[
            jax.ShapeDtypeStruct((NUM_MODELS, IN_DIM, HIDDEN), jnp.bfloat16),
            jax.ShapeDtypeStruct((NUM_MODELS, HIDDEN, HIDDEN), jnp.bfloat16),
            jax.ShapeDtypeStruct((NUM_MODELS, HIDDEN, STATE_DIM), jnp.bfloat16),
        ],
    )(W1, W2, W3)


def _mlp_kernel(te_ref, xp_ref, w1_ref, b1_ref, w2_ref, b2_ref, w3_ref,
                b3_ref, o_ref):
    xb = xp_ref[...].astype(jnp.bfloat16)[:, :IN_DIM]
    h = jnp.maximum(
        jnp.dot(xb, w1_ref[0], preferred_element_type=jnp.float32)
        + b1_ref[0, 0], 0.0).astype(jnp.bfloat16)
    h = jnp.maximum(
        jnp.dot(h, w2_ref[0], preferred_element_type=jnp.float32)
        + b2_ref[0, 0], 0.0).astype(jnp.bfloat16)
    o_ref[...] = (jnp.dot(h, w3_ref[0], preferred_element_type=jnp.float32)
                  + b3_ref[0, 0])


def _grouped_mlp(te, x_pad, W1, b1, W2, b2, W3, b3):
    grid_spec = pltpu.PrefetchScalarGridSpec(
        num_scalar_prefetch=1,
        grid=(T_MAX,),
        in_specs=[
            pl.BlockSpec((R, IN_PAD), lambda t, te: (t, 0)),
            pl.BlockSpec((1, IN_DIM, HIDDEN), lambda t, te: (te[t], 0, 0)),
            pl.BlockSpec((1, 1, HIDDEN), lambda t, te: (te[t], 0, 0)),
            pl.BlockSpec((1, HIDDEN, HIDDEN), lambda t, te: (te[t], 0, 0)),
            pl.BlockSpec((1, 1, HIDDEN), lambda t, te: (te[t], 0, 0)),
            pl.BlockSpec((1, HIDDEN, STATE_DIM), lambda t, te: (te[t], 0, 0)),
            pl.BlockSpec((1, 1, STATE_DIM), lambda t, te: (te[t], 0, 0)),
        ],
        out_specs=pl.BlockSpec((R, STATE_DIM), lambda t, te: (t, 0)),
    )
    return pl.pallas_call(
        _mlp_kernel,
        grid_spec=grid_spec,
        out_shape=jax.ShapeDtypeStruct((PADB, STATE_DIM), jnp.float32),
    )(te, x_pad,
      W1, b1.reshape(NUM_MODELS, 1, HIDDEN),
      W2, b2.reshape(NUM_MODELS, 1, HIDDEN),
      W3, b3.reshape(NUM_MODELS, 1, STATE_DIM))


@jax.jit
def kernel(states, actions, index, W1, b1, W2, b2, W3, b3):
    x = jnp.concatenate(
        [states, actions,
         jnp.zeros((BATCH, IN_PAD - IN_DIM), states.dtype)], axis=-1)
    idx = index.astype(jnp.int32)

    slot, te_c = _routing_meta(idx)

    W1b, W2b, W3b = _cast_weights(W1, W2, W3)
    x_pad = _sc_scatter_rows(x, slot, PADB)
    o_pad = _grouped_mlp(te_c, x_pad, W1b, b1, W2b, b2, W3b, b3)
    return _sc_gather_rows(o_pad, slot)


# speedup vs baseline: 1.0463x; 1.0463x over previous
import jax
import jax.numpy as jnp
from jax.experimental import pallas as pl
from jax.experimental.pallas import tpu as pltpu
from jax.experimental.pallas import tpu_sc as plsc

NUM_MODELS = 8
STATE_DIM = 256
AC_DIM = 64
HIDDEN = 512
BATCH = 4096
IN_DIM = STATE_DIM + AC_DIM

IN_PAD = 384
R = 192
T_MAX = -(-BATCH // R) + NUM_MODELS
PADB = T_MAX * R
WINDOW = 128


def _sc_mesh():
    return plsc.VectorSubcoreMesh(core_axis_name="core",
                                  subcore_axis_name="subcore")


def _sc_scatter_rows(vals, dst, out_rows):
    n, d = vals.shape
    dst2 = dst.reshape(1, n)

    @pl.kernel(out_type=jax.ShapeDtypeStruct((out_rows, d), vals.dtype),
               mesh=_sc_mesh())
    def k(v_hbm, i_hbm, o_hbm):
        def body(v_vmem, i_vmem):
            pltpu.sync_copy(v_vmem, o_hbm.at[i_vmem.at[0]])

        pltpu.emit_pipeline(
            body,
            grid=(n // WINDOW,),
            in_specs=[pl.BlockSpec((WINDOW, d), index_map=lambda i: (i, 0)),
                      pl.BlockSpec((1, WINDOW), index_map=lambda i: (0, i))],
            out_specs=[],
            core_axis_name=("core", "subcore"),
            dimension_semantics=(pltpu.PARALLEL,),
        )(v_hbm, i_hbm)

    return k(vals, dst2)


def _sc_gather_rows(data, g):
    n = g.shape[0]
    d = data.shape[1]
    g2 = g.reshape(1, n)

    @pl.kernel(out_type=jax.ShapeDtypeStruct((n, d), data.dtype),
               mesh=_sc_mesh())
    def k(x_hbm, i_hbm, o_hbm):
        def body(i_vmem, o_vmem):
            pltpu.sync_copy(x_hbm.at[i_vmem.at[0]], o_vmem)

        pltpu.emit_pipeline(
            body,
            grid=(n // WINDOW,),
            in_specs=[pl.BlockSpec((1, WINDOW), index_map=lambda i: (0, i))],
            out_specs=[pl.BlockSpec((WINDOW, d), index_map=lambda i: (i, 0))],
            core_axis_name=("core", "subcore"),
            dimension_semantics=(pltpu.PARALLEL,),
        )(i_hbm, o_hbm)

    return k(data, g2)


_MROWS = 32


def _meta_kernel(idx_ref, slot_ref, te_ref):
    idx2 = idx_ref[...]
    up = (jax.lax.broadcasted_iota(jnp.int32, (128, 128), 0) <
          jax.lax.broadcasted_iota(jnp.int32, (128, 128), 1)
          ).astype(jnp.bfloat16)
    lo = (jax.lax.broadcasted_iota(jnp.int32, (_MROWS, _MROWS), 0) >
          jax.lax.broadcasted_iota(jnp.int32, (_MROWS, _MROWS), 1)
          ).astype(jnp.bfloat16)
    slot = jnp.zeros((_MROWS, 128), jnp.float32)
    ps = jnp.int32(0)
    tile_cum = []
    for e in range(NUM_MODELS):
        m = idx2 == e
        mb = m.astype(jnp.bfloat16)
        lane_pref = jnp.dot(mb, up, preferred_element_type=jnp.float32)
        col_pref = jnp.dot(lo, mb, preferred_element_type=jnp.float32)
        row_off = jnp.sum(col_pref, axis=1, keepdims=True)
        rank = lane_pref + row_off
        count_e = jnp.sum(mb.astype(jnp.float32)).astype(jnp.int32)
        slot = slot + jnp.where(m, rank + ps.astype(jnp.float32), 0.0)
        ps = ps + ((count_e + R - 1) // R) * R
        tile_cum.append(ps // R)
    slot_ref[...] = slot.astype(jnp.int32)
    lane = jax.lax.broadcasted_iota(jnp.int32, (1, 128), 1)
    te = jnp.zeros((1, 128), jnp.int32)
    for e in range(NUM_MODELS):
        te = te + (lane >= tile_cum[e]).astype(jnp.int32)
    te_ref[...] = jnp.minimum(te, NUM_MODELS - 1)


def _routing_meta(idx):
    slot2, te_pad = pl.pallas_call(
        _meta_kernel,
        out_shape=[jax.ShapeDtypeStruct((_MROWS, 128), jnp.int32),
                   jax.ShapeDtypeStruct((1, 128), jnp.int32)],
    )(idx.reshape(_MROWS, 128))
    return slot2.reshape(BATCH), te_pad[0, :T_MAX]


def _mlp_kernel(te_ref, xp_ref, w1_ref, b1_ref, w2_ref, b2_ref, w3_ref,
                b3_ref, o_ref):
    xb = xp_ref[...].astype(jnp.bfloat16)[:, :IN_DIM]
    h = jnp.maximum(
        jnp.dot(xb, w1_ref[0].astype(jnp.bfloat16),
                preferred_element_type=jnp.float32)
        + b1_ref[0, 0], 0.0).astype(jnp.bfloat16)
    h = jnp.maximum(
        jnp.dot(h, w2_ref[0].astype(jnp.bfloat16),
                preferred_element_type=jnp.float32)
        + b2_ref[0, 0], 0.0).astype(jnp.bfloat16)
    o_ref[...] = (jnp.dot(h, w3_ref[0].astype(jnp.bfloat16),
                          preferred_element_type=jnp.float32)
                  + b3_ref[0, 0])


def _grouped_mlp(te, x_pad, W1, b1, W2, b2, W3, b3):
    grid_spec = pltpu.PrefetchScalarGridSpec(
        num_scalar_prefetch=1,
        grid=(T_MAX,),
        in_specs=[
            pl.BlockSpec((R, IN_PAD), lambda t, te: (t, 0)),
            pl.BlockSpec((1, IN_DIM, HIDDEN), lambda t, te: (te[t], 0, 0)),
            pl.BlockSpec((1, 1, HIDDEN), lambda t, te: (te[t], 0, 0)),
            pl.BlockSpec((1, HIDDEN, HIDDEN), lambda t, te: (te[t], 0, 0)),
            pl.BlockSpec((1, 1, HIDDEN), lambda t, te: (te[t], 0, 0)),
            pl.BlockSpec((1, HIDDEN, STATE_DIM), lambda t, te: (te[t], 0, 0)),
            pl.BlockSpec((1, 1, STATE_DIM), lambda t, te: (te[t], 0, 0)),
        ],
        out_specs=pl.BlockSpec((R, STATE_DIM), lambda t, te: (t, 0)),
    )
    return pl.pallas_call(
        _mlp_kernel,
        grid_spec=grid_spec,
        out_shape=jax.ShapeDtypeStruct((PADB, STATE_DIM), jnp.float32),
    )(te, x_pad,
      W1, b1.reshape(NUM_MODELS, 1, HIDDEN),
      W2, b2.reshape(NUM_MODELS, 1, HIDDEN),
      W3, b3.reshape(NUM_MODELS, 1, STATE_DIM))


@jax.jit
def kernel(states, actions, index, W1, b1, W2, b2, W3, b3):
    x = jnp.concatenate(
        [states, actions,
         jnp.zeros((BATCH, IN_PAD - IN_DIM), states.dtype)], axis=-1)
    idx = index.astype(jnp.int32)

    slot, te_c = _routing_meta(idx)

    x_pad = _sc_scatter_rows(x, slot, PADB)
    o_pad = _grouped_mlp(te_c, x_pad, W1, b1, W2, b2, W3, b3)
    return _sc_gather_rows(o_pad, slot)


# speedup vs baseline: 1.1006x; 1.0519x over previous
import jax
import jax.numpy as jnp
from jax.experimental import pallas as pl
from jax.experimental.pallas import tpu as pltpu
from jax.experimental.pallas import tpu_sc as plsc

NUM_MODELS = 8
STATE_DIM = 256
AC_DIM = 64
HIDDEN = 512
BATCH = 4096
IN_DIM = STATE_DIM + AC_DIM

IN_PAD = 384
R = 256
T_MAX = -(-BATCH // R) + NUM_MODELS
PADB = T_MAX * R
WINDOW = 128


def _sc_mesh():
    return plsc.VectorSubcoreMesh(core_axis_name="core",
                                  subcore_axis_name="subcore")


def _sc_scatter_rows(vals, dst, out_rows):
    n, d = vals.shape
    dst2 = dst.reshape(1, n)

    @pl.kernel(out_type=jax.ShapeDtypeStruct((out_rows, d), vals.dtype),
               mesh=_sc_mesh())
    def k(v_hbm, i_hbm, o_hbm):
        def body(v_vmem, i_vmem):
            pltpu.sync_copy(v_vmem, o_hbm.at[i_vmem.at[0]])

        pltpu.emit_pipeline(
            body,
            grid=(n // WINDOW,),
            in_specs=[pl.BlockSpec((WINDOW, d), index_map=lambda i: (i, 0)),
                      pl.BlockSpec((1, WINDOW), index_map=lambda i: (0, i))],
            out_specs=[],
            core_axis_name=("core", "subcore"),
            dimension_semantics=(pltpu.PARALLEL,),
        )(v_hbm, i_hbm)

    return k(vals, dst2)


def _sc_gather_rows(data, g):
    n = g.shape[0]
    d = data.shape[1]
    g2 = g.reshape(1, n)

    @pl.kernel(out_type=jax.ShapeDtypeStruct((n, d), data.dtype),
               mesh=_sc_mesh())
    def k(x_hbm, i_hbm, o_hbm):
        def body(i_vmem, o_vmem):
            pltpu.sync_copy(x_hbm.at[i_vmem.at[0]], o_vmem)

        pltpu.emit_pipeline(
            body,
            grid=(n // WINDOW,),
            in_specs=[pl.BlockSpec((1, WINDOW), index_map=lambda i: (0, i))],
            out_specs=[pl.BlockSpec((WINDOW, d), index_map=lambda i: (i, 0))],
            core_axis_name=("core", "subcore"),
            dimension_semantics=(pltpu.PARALLEL,),
        )(i_hbm, o_hbm)

    return k(data, g2)


_MROWS = 32


def _meta_kernel(idx_ref, slot_ref, te_ref):
    idx2 = idx_ref[...]
    up = (jax.lax.broadcasted_iota(jnp.int32, (128, 128), 0) <
          jax.lax.broadcasted_iota(jnp.int32, (128, 128), 1)
          ).astype(jnp.bfloat16)
    lo = (jax.lax.broadcasted_iota(jnp.int32, (_MROWS, _MROWS), 0) >
          jax.lax.broadcasted_iota(jnp.int32, (_MROWS, _MROWS), 1)
          ).astype(jnp.bfloat16)
    slot = jnp.zeros((_MROWS, 128), jnp.float32)
    ps = jnp.int32(0)
    tile_cum = []
    for e in range(NUM_MODELS):
        m = idx2 == e
        mb = m.astype(jnp.bfloat16)
        lane_pref = jnp.dot(mb, up, preferred_element_type=jnp.float32)
        col_pref = jnp.dot(lo, mb, preferred_element_type=jnp.float32)
        row_off = jnp.sum(col_pref, axis=1, keepdims=True)
        rank = lane_pref + row_off
        count_e = jnp.sum(mb.astype(jnp.float32)).astype(jnp.int32)
        slot = slot + jnp.where(m, rank + ps.astype(jnp.float32), 0.0)
        ps = ps + ((count_e + R - 1) // R) * R
        tile_cum.append(ps // R)
    slot_ref[...] = slot.astype(jnp.int32)
    lane = jax.lax.broadcasted_iota(jnp.int32, (1, 128), 1)
    te = jnp.zeros((1, 128), jnp.int32)
    for e in range(NUM_MODELS):
        te = te + (lane >= tile_cum[e]).astype(jnp.int32)
    te_ref[...] = jnp.minimum(te, NUM_MODELS - 1)


def _routing_meta(idx):
    slot2, te_pad = pl.pallas_call(
        _meta_kernel,
        out_shape=[jax.ShapeDtypeStruct((_MROWS, 128), jnp.int32),
                   jax.ShapeDtypeStruct((1, 128), jnp.int32)],
    )(idx.reshape(_MROWS, 128))
    return slot2.reshape(BATCH), te_pad[0, :T_MAX]


def _mlp_kernel(te_ref, xp_ref, w1_ref, b1_ref, w2_ref, b2_ref, w3_ref,
                b3_ref, o_ref):
    xb = xp_ref[...].astype(jnp.bfloat16)[:, :IN_DIM]
    h = jnp.maximum(
        jnp.dot(xb, w1_ref[0].astype(jnp.bfloat16),
                preferred_element_type=jnp.float32)
        + b1_ref[0, 0], 0.0).astype(jnp.bfloat16)
    h = jnp.maximum(
        jnp.dot(h, w2_ref[0].astype(jnp.bfloat16),
                preferred_element_type=jnp.float32)
        + b2_ref[0, 0], 0.0).astype(jnp.bfloat16)
    o_ref[...] = (jnp.dot(h, w3_ref[0].astype(jnp.bfloat16),
                          preferred_element_type=jnp.float32)
                  + b3_ref[0, 0])


def _grouped_mlp(te, x_pad, W1, b1, W2, b2, W3, b3):
    grid_spec = pltpu.PrefetchScalarGridSpec(
        num_scalar_prefetch=1,
        grid=(T_MAX,),
        in_specs=[
            pl.BlockSpec((R, IN_PAD), lambda t, te: (t, 0)),
            pl.BlockSpec((1, IN_DIM, HIDDEN), lambda t, te: (te[t], 0, 0)),
            pl.BlockSpec((1, 1, HIDDEN), lambda t, te: (te[t], 0, 0)),
            pl.BlockSpec((1, HIDDEN, HIDDEN), lambda t, te: (te[t], 0, 0)),
            pl.BlockSpec((1, 1, HIDDEN), lambda t, te: (te[t], 0, 0)),
            pl.BlockSpec((1, HIDDEN, STATE_DIM), lambda t, te: (te[t], 0, 0)),
            pl.BlockSpec((1, 1, STATE_DIM), lambda t, te: (te[t], 0, 0)),
        ],
        out_specs=pl.BlockSpec((R, STATE_DIM), lambda t, te: (t, 0)),
    )
    return pl.pallas_call(
        _mlp_kernel,
        grid_spec=grid_spec,
        out_shape=jax.ShapeDtypeStruct((PADB, STATE_DIM), jnp.float32),
    )(te, x_pad,
      W1, b1.reshape(NUM_MODELS, 1, HIDDEN),
      W2, b2.reshape(NUM_MODELS, 1, HIDDEN),
      W3, b3.reshape(NUM_MODELS, 1, STATE_DIM))


@jax.jit
def kernel(states, actions, index, W1, b1, W2, b2, W3, b3):
    x = jnp.concatenate(
        [states, actions,
         jnp.zeros((BATCH, IN_PAD - IN_DIM), states.dtype)], axis=-1)
    idx = index.astype(jnp.int32)

    slot, te_c = _routing_meta(idx)

    x_pad = _sc_scatter_rows(x, slot, PADB)
    o_pad = _grouped_mlp(te_c, x_pad, W1, b1, W2, b2, W3, b3)
    return _sc_gather_rows(o_pad, slot)
